# Initial kernel scaffold; baseline (speedup 1.0000x reference)
import functools

import jax
import jax.numpy as jnp
import numpy as np
from jax.experimental import pallas as pl
from jax.experimental.pallas import tpu as pltpu

_ANCHOR_SCALES = (8.0, 16.0, 32.0)
_ANCHOR_RATIOS = (0.5, 1.0, 2.0)
_FEAT_STRIDE = 16
_PRE_NMS = 6000
_POST_NMS = 300
_NMS_THRESH = 0.7
_H = 64
_W = 64
_A = 9
_N_ALL = _H * _W * _A
_N_PAD = 6144
_NB = _N_PAD // 128


def _anchor_ref_np(base_size=16):
    def whctrs(a):
        w = a[2] - a[0] + 1.0
        h = a[3] - a[1] + 1.0
        return w, h, a[0] + 0.5 * (w - 1.0), a[1] + 0.5 * (h - 1.0)

    def mk(ws, hs, xc, yc):
        return np.stack([xc - 0.5 * (ws - 1.0), yc - 0.5 * (hs - 1.0),
                         xc + 0.5 * (ws - 1.0), yc + 0.5 * (hs - 1.0)], axis=1)

    base = np.array([0.0, 0.0, base_size - 1.0, base_size - 1.0])
    w, h, xc, yc = whctrs(base)
    size = w * h
    ratios = np.array(_ANCHOR_RATIOS)
    ws = np.round(np.sqrt(size / ratios))
    hs = np.round(ws * ratios)
    out = []
    for ra in mk(ws, hs, xc, yc):
        w2, h2, xc2, yc2 = whctrs(ra)
        scales = np.array(_ANCHOR_SCALES)
        out.append(mk(w2 * scales, h2 * scales, xc2, yc2))
    return np.concatenate(out, axis=0).astype(np.float32)


def _anchors_np():
    ref = _anchor_ref_np()
    sx = np.arange(_W) * _FEAT_STRIDE
    sy = np.arange(_H) * _FEAT_STRIDE
    SX, SY = np.meshgrid(sx, sy)
    shifts = np.stack([SX.ravel(), SY.ravel(), SX.ravel(), SY.ravel()],
                      axis=1).astype(np.float32)
    return (shifts[:, None, :] + ref[None, :, :]).reshape(-1, 4)


_ANC = _anchors_np()
_AW = (_ANC[:, 2] - _ANC[:, 0] + np.float32(1.0)).reshape(_H * _W, _A)
_AH = (_ANC[:, 3] - _ANC[:, 1] + np.float32(1.0)).reshape(_H * _W, _A)
_ACX = (_ANC[:, 0] + np.float32(0.5) * _AW.reshape(-1)).reshape(_H * _W, _A)
_ACY = (_ANC[:, 1] + np.float32(0.5) * _AH.reshape(-1)).reshape(_H * _W, _A)

_ROWS_PER_STEP = 8
_GRID = _H // _ROWS_PER_STEP
_RP = _ROWS_PER_STEP * _W


def _dense_body(hi_ref, xp_ref, xc_ref, xn_ref, w1_ref, wh_ref, b1_ref,
                bh_ref, aw_ref, ah_ref, acx_ref, acy_ref,
                sc_out, x1_out, y1_out, x2_out, y2_out):
    r = pl.program_id(0)
    xp = xp_ref[...]
    xc = xc_ref[...]
    xn = xn_ref[...]
    top = xp[7:8] * (r > 0).astype(jnp.float32)
    bot = xn[0:1] * (r < _GRID - 1).astype(jnp.float32)
    rows = jnp.concatenate([top, xc, bot], axis=0)

    zcol = jnp.zeros((_ROWS_PER_STEP, 1, 512), jnp.float32)
    acc = jnp.zeros((_RP, 512), jnp.float32)
    for dy in range(3):
        xs = rows[dy:dy + _ROWS_PER_STEP]
        left = jnp.concatenate([zcol, xs[:, :-1]], axis=1)
        right = jnp.concatenate([xs[:, 1:], zcol], axis=1)
        xcat = jnp.concatenate([left, xs, right], axis=2)
        xcat = xcat.reshape(_RP, 3 * 512)
        acc = acc + jnp.dot(xcat, w1_ref[dy],
                            preferred_element_type=jnp.float32,
                            precision=jax.lax.Precision.HIGHEST)
    y = jnp.maximum(acc + b1_ref[...], 0.0)
    s = jnp.dot(y, wh_ref[...], preferred_element_type=jnp.float32,
                precision=jax.lax.Precision.HIGHEST) + bh_ref[...]
    s0 = s[:, 0:9]
    s1 = s[:, 9:18]
    dx = s[:, 18:27]
    dyy = s[:, 27:36]
    dw = s[:, 36:45]
    dh = s[:, 45:54]
    m = jnp.maximum(s0, s1)
    e0 = jnp.exp(s0 - m)
    e1 = jnp.exp(s1 - m)
    sc_out[...] = e1 / (e0 + e1)

    aw = aw_ref[...]
    ah = ah_ref[...]
    pcx = dx * aw + acx_ref[...]
    pcy = dyy * ah + acy_ref[...]
    pw = jnp.exp(dw) * aw
    ph = jnp.exp(dh) * ah
    hi = hi_ref[0]
    x1_out[...] = jnp.clip(pcx - 0.5 * pw, 0.0, hi)
    y1_out[...] = jnp.clip(pcy - 0.5 * ph, 0.0, hi)
    x2_out[...] = jnp.clip(pcx + 0.5 * pw, 0.0, hi)
    y2_out[...] = jnp.clip(pcy + 0.5 * ph, 0.0, hi)


def _dense_call(x, w1r, whead, b1, bhead, clip_hi):
    grid = (_GRID,)
    rows_spec = lambda off: pl.BlockSpec(
        (_ROWS_PER_STEP, _W, 512),
        lambda r, off=off: (jnp.clip(r + off, 0, _GRID - 1), 0, 0))
    pos_spec = pl.BlockSpec((_RP, _A), lambda r: (r, 0))
    out = pl.pallas_call(
        _dense_body,
        grid=grid,
        in_specs=[
            pl.BlockSpec(memory_space=pltpu.SMEM),
            rows_spec(-1), rows_spec(0), rows_spec(1),
            pl.BlockSpec((3, 3 * 512, 512), lambda r: (0, 0, 0)),
            pl.BlockSpec((512, 54), lambda r: (0, 0)),
            pl.BlockSpec((1, 512), lambda r: (0, 0)),
            pl.BlockSpec((1, 54), lambda r: (0, 0)),
            pos_spec, pos_spec, pos_spec, pos_spec,
        ],
        out_specs=[pos_spec] * 5,
        out_shape=[jax.ShapeDtypeStruct((_H * _W, _A), jnp.float32)] * 5,
    )
    return out(clip_hi, x, x, x, w1r, whead, b1, bhead,
               jnp.asarray(_AW), jnp.asarray(_AH),
               jnp.asarray(_ACX), jnp.asarray(_ACY))


def _nms_body(x1r_ref, y1r_ref, x2r_ref, y2r_ref,
              x1c_ref, y1c_ref, x2c_ref, y2c_ref,
              keep_out, sup_ref, m_ref, arear_ref, areac_ref):
    arear_ref[...] = ((x2r_ref[...] - x1r_ref[...] + 1.0) *
                      (y2r_ref[...] - y1r_ref[...] + 1.0))
    areac_ref[...] = ((x2c_ref[...] - x1c_ref[...] + 1.0) *
                      (y2c_ref[...] - y1c_ref[...] + 1.0))
    ridx = jax.lax.broadcasted_iota(jnp.int32, (_NB, 128), 0)
    cidx = jax.lax.broadcasted_iota(jnp.int32, (_NB, 128), 1)
    sup_ref[...] = ((ridx * 128 + cidx) >= _PRE_NMS).astype(jnp.float32)

    lane = jax.lax.broadcasted_iota(jnp.int32, (1, 128), 1)
    sub_i = jax.lax.broadcasted_iota(jnp.int32, (128, 128), 0)
    lane_j = jax.lax.broadcasted_iota(jnp.int32, (128, 128), 1)

    def iou_mat(k, c):
        x1i = x1c_ref[pl.ds(k * 128, 128), :]
        y1i = y1c_ref[pl.ds(k * 128, 128), :]
        x2i = x2c_ref[pl.ds(k * 128, 128), :]
        y2i = y2c_ref[pl.ds(k * 128, 128), :]
        ai = areac_ref[pl.ds(k * 128, 128), :]
        x1j = x1r_ref[pl.ds(c, 1), :]
        y1j = y1r_ref[pl.ds(c, 1), :]
        x2j = x2r_ref[pl.ds(c, 1), :]
        y2j = y2r_ref[pl.ds(c, 1), :]
        aj = arear_ref[pl.ds(c, 1), :]
        xx1 = jnp.maximum(x1i, x1j)
        yy1 = jnp.maximum(y1i, y1j)
        xx2 = jnp.minimum(x2i, x2j)
        yy2 = jnp.minimum(y2i, y2j)
        ww = jnp.maximum(0.0, xx2 - xx1 + 1.0)
        hh = jnp.maximum(0.0, yy2 - yy1 + 1.0)
        inter = ww * hh
        iou = inter / (ai + aj - inter)
        return iou > _NMS_THRESH

    def block_step(k, _):
        m_ref[...] = jnp.where(iou_mat(k, k) & (lane_j > sub_i), 1.0, 0.0)

        def inner(i, supb):
            oh = (lane == i).astype(jnp.float32)
            cur = jnp.sum(supb * oh)
            row = m_ref[pl.ds(i, 1), :]
            live = (cur < 0.5).astype(jnp.float32)
            return jnp.maximum(supb, row * live)

        supb = jax.lax.fori_loop(0, 128, inner,
                                 sup_ref[pl.ds(k, 1), :])
        sup_ref[pl.ds(k, 1), :] = supb
        kept = 1.0 - supb

        def cross(c, _):
            a = jnp.where(iou_mat(k, c), 1.0, 0.0)
            hit = jnp.dot(kept, a, preferred_element_type=jnp.float32)
            upd = (hit > 0.5).astype(jnp.float32)
            sup_ref[pl.ds(c, 1), :] = jnp.maximum(sup_ref[pl.ds(c, 1), :],
                                                  upd)
            return 0

        jax.lax.fori_loop(k + 1, _NB, cross, 0)
        return 0

    jax.lax.fori_loop(0, _NB, block_step, 0)
    keep_out[...] = 1.0 - sup_ref[...]


def _nms_call(bx1, by1, bx2, by2):
    rview = lambda v: v.reshape(_NB, 128)
    cview = lambda v: v.reshape(_N_PAD, 1)
    keep = pl.pallas_call(
        _nms_body,
        out_shape=jax.ShapeDtypeStruct((_NB, 128), jnp.float32),
        scratch_shapes=[
            pltpu.VMEM((_NB, 128), jnp.float32),
            pltpu.VMEM((128, 128), jnp.float32),
            pltpu.VMEM((_NB, 128), jnp.float32),
            pltpu.VMEM((_N_PAD, 1), jnp.float32),
        ],
    )(rview(bx1), rview(by1), rview(bx2), rview(by2),
      cview(bx1), cview(by1), cview(bx2), cview(by2))
    return keep.reshape(_N_PAD)


def kernel(feature_map, img_size, W1, b1, Ws, bs, Wb, bb):
    x = feature_map.reshape(_H, _W, 512)
    w1r = W1.reshape(3, 3 * 512, 512)
    ws2 = Ws.reshape(512, 18)
    wb2 = Wb.reshape(512, 36)
    whead = jnp.concatenate(
        [ws2[:, 0::2], ws2[:, 1::2],
         wb2[:, 0::4], wb2[:, 1::4], wb2[:, 2::4], wb2[:, 3::4]], axis=1)
    bhead = jnp.concatenate(
        [bs[0::2], bs[1::2], bb[0::4], bb[1::4], bb[2::4], bb[3::4]]
    ).reshape(1, 54)
    clip_hi = (jnp.asarray(img_size, jnp.float32) - 1.0).reshape(1)

    sc, px1, py1, px2, py2 = _dense_call(x, w1r, whead, b1.reshape(1, 512),
                                         bhead, clip_hi)
    scores = sc.reshape(-1)

    top_scores, order = jax.lax.top_k(scores, _PRE_NMS)
    orderp = jnp.concatenate(
        [order, jnp.zeros((_N_PAD - _PRE_NMS,), order.dtype)])
    bx1 = px1.reshape(-1)[orderp]
    by1 = py1.reshape(-1)[orderp]
    bx2 = px2.reshape(-1)[orderp]
    by2 = py2.reshape(-1)[orderp]

    keepf = _nms_call(bx1, by1, bx2, by2)
    keep = keepf[:_PRE_NMS] > 0.5

    masked = jnp.where(keep, top_scores, -jnp.inf)
    _, idx = jax.lax.top_k(masked, _POST_NMS)
    rois = jnp.stack([bx1[idx], by1[idx], bx2[idx], by2[idx]], axis=1)
    return rois


# speedup vs baseline: 50.1251x; 50.1251x over previous
import functools

import jax
import jax.numpy as jnp
import numpy as np
from jax.experimental import pallas as pl
from jax.experimental.pallas import tpu as pltpu

_ANCHOR_SCALES = (8.0, 16.0, 32.0)
_ANCHOR_RATIOS = (0.5, 1.0, 2.0)
_FEAT_STRIDE = 16
_PRE_NMS = 6000
_POST_NMS = 300
_NMS_THRESH = 0.7
_H = 64
_W = 64
_A = 9
_N_ALL = _H * _W * _A
_N_PAD = 6144
_NB = _N_PAD // 128


def _anchor_ref_np(base_size=16):
    def whctrs(a):
        w = a[2] - a[0] + 1.0
        h = a[3] - a[1] + 1.0
        return w, h, a[0] + 0.5 * (w - 1.0), a[1] + 0.5 * (h - 1.0)

    def mk(ws, hs, xc, yc):
        return np.stack([xc - 0.5 * (ws - 1.0), yc - 0.5 * (hs - 1.0),
                         xc + 0.5 * (ws - 1.0), yc + 0.5 * (hs - 1.0)], axis=1)

    base = np.array([0.0, 0.0, base_size - 1.0, base_size - 1.0])
    w, h, xc, yc = whctrs(base)
    size = w * h
    ratios = np.array(_ANCHOR_RATIOS)
    ws = np.round(np.sqrt(size / ratios))
    hs = np.round(ws * ratios)
    out = []
    for ra in mk(ws, hs, xc, yc):
        w2, h2, xc2, yc2 = whctrs(ra)
        scales = np.array(_ANCHOR_SCALES)
        out.append(mk(w2 * scales, h2 * scales, xc2, yc2))
    return np.concatenate(out, axis=0).astype(np.float32)


def _anchors_np():
    ref = _anchor_ref_np()
    sx = np.arange(_W) * _FEAT_STRIDE
    sy = np.arange(_H) * _FEAT_STRIDE
    SX, SY = np.meshgrid(sx, sy)
    shifts = np.stack([SX.ravel(), SY.ravel(), SX.ravel(), SY.ravel()],
                      axis=1).astype(np.float32)
    return (shifts[:, None, :] + ref[None, :, :]).reshape(-1, 4)


_ANC = _anchors_np()
_AW = (_ANC[:, 2] - _ANC[:, 0] + np.float32(1.0)).reshape(_H * _W, _A)
_AH = (_ANC[:, 3] - _ANC[:, 1] + np.float32(1.0)).reshape(_H * _W, _A)
_ACX = (_ANC[:, 0] + np.float32(0.5) * _AW.reshape(-1)).reshape(_H * _W, _A)
_ACY = (_ANC[:, 1] + np.float32(0.5) * _AH.reshape(-1)).reshape(_H * _W, _A)

_ROWS_PER_STEP = 8
_GRID = _H // _ROWS_PER_STEP
_RP = _ROWS_PER_STEP * _W


def _dense_body(hi_ref, xp_ref, xc_ref, xn_ref, w1_ref, wh_ref, b1_ref,
                bh_ref, aw_ref, ah_ref, acx_ref, acy_ref,
                sc_out, x1_out, y1_out, x2_out, y2_out):
    r = pl.program_id(0)
    xp = xp_ref[...]
    xc = xc_ref[...]
    xn = xn_ref[...]
    top = xp[7:8] * (r > 0).astype(jnp.float32)
    bot = xn[0:1] * (r < _GRID - 1).astype(jnp.float32)
    rows = jnp.concatenate([top, xc, bot], axis=0)
    rows = rows.astype(jnp.bfloat16).astype(jnp.float32)

    zcol = jnp.zeros((_ROWS_PER_STEP, 1, 512), jnp.float32)
    acc = None
    for dy in range(3):
        xs = rows[dy:dy + _ROWS_PER_STEP]
        left = jnp.concatenate([zcol, xs[:, :-1]], axis=1)
        right = jnp.concatenate([xs[:, 1:], zcol], axis=1)
        for dx, xt in enumerate((left, xs, right)):
            xt = xt.reshape(_RP, 512)
            for c in range(2):
                t = jnp.dot(xt[:, c * 256:(c + 1) * 256],
                            w1_ref[pl.ds((dy * 3 + dx) * 512 + c * 256, 256),
                                   :],
                            preferred_element_type=jnp.float32)
                acc = t if acc is None else acc + t
    y = jnp.maximum(acc + b1_ref[...], 0.0)
    yb = y.astype(jnp.bfloat16).astype(jnp.float32)
    s = jnp.dot(yb, wh_ref[...],
                preferred_element_type=jnp.float32) + bh_ref[...]
    s0 = s[:, 0:9]
    s1 = s[:, 9:18]
    dx = s[:, 18:27]
    dyy = s[:, 27:36]
    dw = s[:, 36:45]
    dh = s[:, 45:54]
    m = jnp.maximum(s0, s1)
    e0 = jnp.exp(s0 - m)
    e1 = jnp.exp(s1 - m)
    sc_out[...] = e1 / (e0 + e1)

    aw = aw_ref[...]
    ah = ah_ref[...]
    pcx = dx * aw + acx_ref[...]
    pcy = dyy * ah + acy_ref[...]
    pw = jnp.exp(dw) * aw
    ph = jnp.exp(dh) * ah
    hi = hi_ref[0]
    x1_out[...] = jnp.clip(pcx - 0.5 * pw, 0.0, hi)
    y1_out[...] = jnp.clip(pcy - 0.5 * ph, 0.0, hi)
    x2_out[...] = jnp.clip(pcx + 0.5 * pw, 0.0, hi)
    y2_out[...] = jnp.clip(pcy + 0.5 * ph, 0.0, hi)


def _dense_call(x, w1r, whead, b1, bhead, clip_hi):
    grid = (_GRID,)
    rows_spec = lambda off: pl.BlockSpec(
        (_ROWS_PER_STEP, _W, 512),
        lambda r, off=off: (jnp.clip(r + off, 0, _GRID - 1), 0, 0))
    pos_spec = pl.BlockSpec((_RP, _A), lambda r: (r, 0))
    out = pl.pallas_call(
        _dense_body,
        grid=grid,
        in_specs=[
            pl.BlockSpec(memory_space=pltpu.SMEM),
            rows_spec(-1), rows_spec(0), rows_spec(1),
            pl.BlockSpec((9 * 512, 512), lambda r: (0, 0)),
            pl.BlockSpec((512, 54), lambda r: (0, 0)),
            pl.BlockSpec((1, 512), lambda r: (0, 0)),
            pl.BlockSpec((1, 54), lambda r: (0, 0)),
            pos_spec, pos_spec, pos_spec, pos_spec,
        ],
        out_specs=[pos_spec] * 5,
        out_shape=[jax.ShapeDtypeStruct((_H * _W, _A), jnp.float32)] * 5,
    )
    return out(clip_hi, x, x, x, w1r, whead, b1, bhead,
               jnp.asarray(_AW), jnp.asarray(_AH),
               jnp.asarray(_ACX), jnp.asarray(_ACY))


def _nms_body(x1r_ref, y1r_ref, x2r_ref, y2r_ref,
              x1c_ref, y1c_ref, x2c_ref, y2c_ref,
              keep_out, sup_ref, m_ref, arear_ref, areac_ref):
    arear_ref[...] = ((x2r_ref[...] - x1r_ref[...] + 1.0) *
                      (y2r_ref[...] - y1r_ref[...] + 1.0))
    areac_ref[...] = ((x2c_ref[...] - x1c_ref[...] + 1.0) *
                      (y2c_ref[...] - y1c_ref[...] + 1.0))
    ridx = jax.lax.broadcasted_iota(jnp.int32, (_NB, 128), 0)
    cidx = jax.lax.broadcasted_iota(jnp.int32, (_NB, 128), 1)
    sup_ref[...] = ((ridx * 128 + cidx) >= _PRE_NMS).astype(jnp.float32)

    lane = jax.lax.broadcasted_iota(jnp.int32, (1, 128), 1)
    sub_i = jax.lax.broadcasted_iota(jnp.int32, (128, 128), 0)
    lane_j = jax.lax.broadcasted_iota(jnp.int32, (128, 128), 1)

    def iou_mat(k, c):
        x1i = x1c_ref[pl.ds(k * 128, 128), :]
        y1i = y1c_ref[pl.ds(k * 128, 128), :]
        x2i = x2c_ref[pl.ds(k * 128, 128), :]
        y2i = y2c_ref[pl.ds(k * 128, 128), :]
        ai = areac_ref[pl.ds(k * 128, 128), :]
        x1j = x1r_ref[pl.ds(c, 1), :]
        y1j = y1r_ref[pl.ds(c, 1), :]
        x2j = x2r_ref[pl.ds(c, 1), :]
        y2j = y2r_ref[pl.ds(c, 1), :]
        aj = arear_ref[pl.ds(c, 1), :]
        xx1 = jnp.maximum(x1i, x1j)
        yy1 = jnp.maximum(y1i, y1j)
        xx2 = jnp.minimum(x2i, x2j)
        yy2 = jnp.minimum(y2i, y2j)
        ww = jnp.maximum(0.0, xx2 - xx1 + 1.0)
        hh = jnp.maximum(0.0, yy2 - yy1 + 1.0)
        inter = ww * hh
        iou = inter / (ai + aj - inter)
        return iou > _NMS_THRESH

    def block_step(k, _):
        m_ref[...] = jnp.where(iou_mat(k, k) & (lane_j > sub_i), 1.0, 0.0)

        def inner(i, supb):
            oh = (lane == i).astype(jnp.float32)
            cur = jnp.sum(supb * oh)
            row = m_ref[pl.ds(i, 1), :]
            live = (cur < 0.5).astype(jnp.float32)
            return jnp.maximum(supb, row * live)

        supb = jax.lax.fori_loop(0, 128, inner,
                                 sup_ref[pl.ds(k, 1), :])
        sup_ref[pl.ds(k, 1), :] = supb
        kept = 1.0 - supb

        def cross(c, _):
            a = jnp.where(iou_mat(k, c), 1.0, 0.0)
            hit = jnp.dot(kept, a, preferred_element_type=jnp.float32)
            upd = (hit > 0.5).astype(jnp.float32)
            sup_ref[pl.ds(c, 1), :] = jnp.maximum(sup_ref[pl.ds(c, 1), :],
                                                  upd)
            return 0

        jax.lax.fori_loop(k + 1, _NB, cross, 0)
        return 0

    jax.lax.fori_loop(0, _NB, block_step, 0)
    keep_out[...] = 1.0 - sup_ref[...]


def _nms_call(bx1, by1, bx2, by2):
    rview = lambda v: v.reshape(_NB, 128)
    cview = lambda v: v.reshape(_N_PAD, 1)
    keep = pl.pallas_call(
        _nms_body,
        out_shape=jax.ShapeDtypeStruct((_NB, 128), jnp.float32),
        scratch_shapes=[
            pltpu.VMEM((_NB, 128), jnp.float32),
            pltpu.VMEM((128, 128), jnp.float32),
            pltpu.VMEM((_NB, 128), jnp.float32),
            pltpu.VMEM((_N_PAD, 1), jnp.float32),
        ],
    )(rview(bx1), rview(by1), rview(bx2), rview(by2),
      cview(bx1), cview(by1), cview(bx2), cview(by2))
    return keep.reshape(_N_PAD)


def kernel(feature_map, img_size, W1, b1, Ws, bs, Wb, bb):
    bfv = lambda a: a.astype(jnp.bfloat16).astype(jnp.float32)
    x = feature_map.reshape(_H, _W, 512)
    w1r = bfv(W1.reshape(9 * 512, 512))
    ws2 = Ws.reshape(512, 18)
    wb2 = Wb.reshape(512, 36)
    whead = bfv(jnp.concatenate(
        [ws2[:, 0::2], ws2[:, 1::2],
         wb2[:, 0::4], wb2[:, 1::4], wb2[:, 2::4], wb2[:, 3::4]], axis=1))
    bhead = jnp.concatenate(
        [bs[0::2], bs[1::2], bb[0::4], bb[1::4], bb[2::4], bb[3::4]]
    ).reshape(1, 54)
    clip_hi = (jnp.asarray(img_size, jnp.float32) - 1.0).reshape(1)

    sc, px1, py1, px2, py2 = _dense_call(x, w1r, whead, b1.reshape(1, 512),
                                         bhead, clip_hi)
    scores = sc.reshape(-1)

    top_scores, order = jax.lax.top_k(scores, _PRE_NMS)
    orderp = jnp.concatenate(
        [order, jnp.zeros((_N_PAD - _PRE_NMS,), order.dtype)])
    bx1 = px1.reshape(-1)[orderp]
    by1 = py1.reshape(-1)[orderp]
    bx2 = px2.reshape(-1)[orderp]
    by2 = py2.reshape(-1)[orderp]

    keepf = _nms_call(bx1, by1, bx2, by2)
    keep = keepf[:_PRE_NMS] > 0.5

    masked = jnp.where(keep, top_scores, -jnp.inf)
    _, idx = jax.lax.top_k(masked, _POST_NMS)
    rois = jnp.stack([bx1[idx], by1[idx], bx2[idx], by2[idx]], axis=1)
    return rois
